# baseline (speedup 1.0000x reference)
import functools

import jax
import jax.numpy as jnp
from jax import lax
from jax.experimental import pallas as pl
from jax.experimental.pallas import tpu as pltpu
from jax.experimental.pallas import tpu_sc as plsc

_N = 10000
_E = 320000
_F_IN = 128
_H = 64
_C = 6

_NC, _NS = 2, 16
_CHUNK = 128
_CPT0 = 32
_CPT1 = 128
_CPTMAX = max(_CPT0, _CPT1)
_NCHUNKS_PAD = _NS * (_CPT0 + _CPT1)
_EPAD = _NCHUNKS_PAD * _CHUNK
_DUMP = 10000
_ROWS_PAD = 10240
_RPT = _ROWS_PAD // _NS
_NBUF = 4
_HBINS = 16384
_HSTRIP = _HBINS // _NS

_mesh = plsc.VectorSubcoreMesh(
    core_axis_name="c", subcore_axis_name="s",
    num_cores=_NC, num_subcores=_NS)


@functools.partial(
    pl.kernel,
    out_type=jax.ShapeDtypeStruct((_NC, _HBINS // 128, 128), jnp.float32),
    mesh=_mesh,
    scratch_types=[
        pltpu.VMEM((_CPTMAX, _CHUNK), jnp.int32),
        pltpu.VMEM((_HBINS,), jnp.float32),
        pltpu.VMEM((_HBINS,), jnp.float32),
        pltpu.VMEM((_HSTRIP // 128, 128), jnp.float32),
        pltpu.VMEM_SHARED((_NS, _HBINS), jnp.float32),
    ],
    compiler_params=pltpu.CompilerParams(needs_layout_passes=False),
)
def _sc_degree(edge_hbm, zerosh_hbm, out_hbm, idx_v, hist_v, buf_v, res_v,
               acc_sh):
    cid = lax.axis_index("c")
    sid = lax.axis_index("s")
    pltpu.sync_copy(zerosh_hbm, hist_v)
    ones16 = jnp.full((16,), 1.0, jnp.float32)

    def hist_part(cpt, base):
        pltpu.sync_copy(edge_hbm.at[1, pl.ds(base, cpt)],
                        idx_v.at[pl.ds(0, cpt)])
        for j in range(cpt):
            for k in range(_CHUNK // 16):
                d = idx_v[j, pl.ds(k * 16, 16)]
                plsc.addupdate_scatter(hist_v, [d], ones16)

    @pl.when(cid == 0)
    def _():
        hist_part(_CPT0, pl.multiple_of(sid * _CPT0, 8))

    if _CPT1 > 0:
        @pl.when(cid == 1)
        def _():
            hist_part(_CPT1,
                      pl.multiple_of(_NS * _CPT0 + sid * _CPT1, 8))

    pltpu.sync_copy(hist_v, acc_sh.at[sid])
    plsc.subcore_barrier()
    for r in range(_NS):
        pltpu.sync_copy(acc_sh.at[r, pl.ds(sid * _HSTRIP, _HSTRIP)],
                        buf_v.at[pl.ds(r * _HSTRIP, _HSTRIP)])
    for c in range(_HSTRIP // 16):
        acc = buf_v[pl.ds(c * 16, 16)]
        for r in range(1, _NS):
            acc = acc + buf_v[pl.ds(r * _HSTRIP + c * 16, 16)]
        res_v[c // 8, pl.ds((c % 8) * 16, 16)] = acc
    pltpu.sync_copy(res_v,
                    out_hbm.at[cid, pl.ds(sid * (_HSTRIP // 128),
                                          _HSTRIP // 128)])


@functools.partial(
    pl.kernel,
    out_type=jax.ShapeDtypeStruct((_NC, _ROWS_PAD, _H), jnp.float32),
    mesh=_mesh,
    scratch_types=(
        [pltpu.VMEM((_CPTMAX, _CHUNK), jnp.int32),
         pltpu.VMEM((_CPTMAX, _CHUNK), jnp.int32)]
        + [pltpu.VMEM((_CHUNK, _H), jnp.float32)] * _NBUF
        + [pltpu.SemaphoreType.DMA] * (2 * _NBUF)
        + [pltpu.VMEM_SHARED((_ROWS_PAD, _H), jnp.float32)]
    ),
    compiler_params=pltpu.CompilerParams(use_tc_tiling_on_sc=False),
)
def _sc_scatter(edge_hbm, g_hbm, zeros2_hbm, out_hbm, *rest):
    src_v, dst_v = rest[0], rest[1]
    rows = rest[2:2 + _NBUF]
    gsem = rest[2 + _NBUF:2 + 2 * _NBUF]
    ssem = rest[2 + 2 * _NBUF:2 + 3 * _NBUF]
    acc_sh = rest[2 + 3 * _NBUF]
    cid = lax.axis_index("c")
    sid = lax.axis_index("s")
    pltpu.sync_copy(zeros2_hbm, acc_sh.at[pl.ds(sid * _RPT, _RPT)])
    plsc.subcore_barrier()

    def ring(cpt, base):
        pltpu.sync_copy(edge_hbm.at[0, pl.ds(base, cpt)],
                        src_v.at[pl.ds(0, cpt)])
        pltpu.sync_copy(edge_hbm.at[1, pl.ds(base, cpt)],
                        dst_v.at[pl.ds(0, cpt)])
        gh = [None] * cpt
        sh = [None] * cpt
        for j in range(_NBUF - 1):
            gh[j] = pltpu.async_copy(g_hbm.at[src_v.at[j]], rows[j % _NBUF],
                                     gsem[j % _NBUF])
        for j in range(cpt):
            b = j % _NBUF
            jn = j + _NBUF - 1
            if jn < cpt:
                bn = jn % _NBUF
                if jn >= _NBUF:
                    sh[jn - _NBUF].wait()
                gh[jn] = pltpu.async_copy(g_hbm.at[src_v.at[jn]], rows[bn],
                                          gsem[bn])
            gh[j].wait()
            sh[j] = pltpu.async_copy(rows[b], acc_sh.at[dst_v.at[j]],
                                     ssem[b], add=True)
        for j in range(max(cpt - _NBUF, 0), cpt):
            sh[j].wait()

    @pl.when(cid == 0)
    def _():
        ring(_CPT0, pl.multiple_of(sid * _CPT0, 8))

    if _CPT1 > 0:
        @pl.when(cid == 1)
        def _():
            ring(_CPT1, pl.multiple_of(_NS * _CPT0 + sid * _CPT1, 8))

    plsc.subcore_barrier()
    pltpu.sync_copy(acc_sh.at[pl.ds(sid * _RPT, _RPT)],
                    out_hbm.at[cid, pl.ds(sid * _RPT, _RPT)])


_BLK = 1000
_GRID = _N // _BLK


def _stage1_body(degp_ref, x_ref, w1_ref, g1_ref, dis_ref):
    deg = degp_ref[:, 0:1] + degp_ref[:, 1:2] + 1.0
    dis = lax.rsqrt(deg)
    h = jnp.dot(x_ref[...], w1_ref[...], preferred_element_type=jnp.float32)
    g1_ref[...] = h * dis
    dis_ref[...] = dis


def _stage1(degp, x, w1):
    return pl.pallas_call(
        _stage1_body,
        grid=(_GRID,),
        in_specs=[
            pl.BlockSpec((_BLK, 2), lambda i: (i, 0)),
            pl.BlockSpec((_BLK, _F_IN), lambda i: (i, 0)),
            pl.BlockSpec((_F_IN, _H), lambda i: (0, 0)),
        ],
        out_specs=[
            pl.BlockSpec((_BLK, _H), lambda i: (i, 0)),
            pl.BlockSpec((_BLK, 1), lambda i: (i, 0)),
        ],
        out_shape=[
            jax.ShapeDtypeStruct((_N, _H), jnp.float32),
            jax.ShapeDtypeStruct((_N, 1), jnp.float32),
        ],
    )(degp, x, w1)


def _stage2_body(q_ref, g1_ref, dis_ref, b1_ref, w2_ref, g2_ref):
    s = q_ref[0] + q_ref[1] + g1_ref[...]
    h = jnp.maximum(s * dis_ref[...] + b1_ref[...], 0.0)
    h2 = jnp.dot(h, w2_ref[...], preferred_element_type=jnp.float32)
    g2_ref[...] = h2 * dis_ref[...]


def _stage2(q, g1, dis, b1, w2):
    return pl.pallas_call(
        _stage2_body,
        grid=(_GRID,),
        in_specs=[
            pl.BlockSpec((2, _BLK, _H), lambda i: (0, i, 0)),
            pl.BlockSpec((_BLK, _H), lambda i: (i, 0)),
            pl.BlockSpec((_BLK, 1), lambda i: (i, 0)),
            pl.BlockSpec((1, _H), lambda i: (0, 0)),
            pl.BlockSpec((_H, _H), lambda i: (0, 0)),
        ],
        out_specs=pl.BlockSpec((_BLK, _H), lambda i: (i, 0)),
        out_shape=jax.ShapeDtypeStruct((_N, _H), jnp.float32),
    )(q, g1, dis, b1, w2)


def _stage3_body(r_ref, g2_ref, dis_ref, b2_ref, wp_ref, bp_ref,
                 wc_ref, bc_ref, logits_ref, z_ref):
    s = r_ref[0] + r_ref[1] + g2_ref[...]
    h = jnp.maximum(s * dis_ref[...] + b2_ref[...], 0.0)
    z = jnp.maximum(
        jnp.dot(h, wp_ref[...], preferred_element_type=jnp.float32)
        + bp_ref[...], 0.0)
    z_ref[...] = z
    logits_ref[...] = (
        jnp.dot(z, wc_ref[...], preferred_element_type=jnp.float32)
        + bc_ref[...])


def _stage3(r, g2, dis, b2, wp, bp, wc, bc):
    return pl.pallas_call(
        _stage3_body,
        grid=(_GRID,),
        in_specs=[
            pl.BlockSpec((2, _BLK, _H), lambda i: (0, i, 0)),
            pl.BlockSpec((_BLK, _H), lambda i: (i, 0)),
            pl.BlockSpec((_BLK, 1), lambda i: (i, 0)),
            pl.BlockSpec((1, _H), lambda i: (0, 0)),
            pl.BlockSpec((_H, _H), lambda i: (0, 0)),
            pl.BlockSpec((1, _H), lambda i: (0, 0)),
            pl.BlockSpec((_H, _C), lambda i: (0, 0)),
            pl.BlockSpec((1, _C), lambda i: (0, 0)),
        ],
        out_specs=[
            pl.BlockSpec((_BLK, _C), lambda i: (i, 0)),
            pl.BlockSpec((_BLK, _H), lambda i: (i, 0)),
        ],
        out_shape=[
            jax.ShapeDtypeStruct((_N, _C), jnp.float32),
            jax.ShapeDtypeStruct((_N, _H), jnp.float32),
        ],
    )(r, g2, dis, b2, wp, bp, wc, bc)


def kernel(x, edge_index, W1, b1, W2, b2, Wp, bp, Wc, bc):
    zerosh = jnp.zeros((_HBINS,), jnp.float32)
    zeros2 = jnp.zeros((_RPT, _H), jnp.float32)

    pad = jnp.stack(
        [jnp.zeros((_EPAD - _E,), jnp.int32),
         _DUMP + (jnp.arange(_EPAD - _E, dtype=jnp.int32) % 240)], axis=0)
    e3 = jnp.concatenate([edge_index, pad], axis=1).reshape(
        2, _NCHUNKS_PAD, _CHUNK)

    degp = _sc_degree(e3, zerosh)
    degp = degp.reshape(_NC, _HBINS)[:, :_N].T

    g1, dis = _stage1(degp, x, W1)

    q = _sc_scatter(e3, g1, zeros2)
    g2 = _stage2(q, g1, dis, b1.reshape(1, _H), W2)

    r = _sc_scatter(e3, g2, zeros2)
    logits, z = _stage3(r, g2, dis, b2.reshape(1, _H), Wp,
                        bp.reshape(1, _H), Wc, bc.reshape(1, _C))
    return (logits, z)


# speedup vs baseline: 1.2443x; 1.2443x over previous
import functools

import jax
import jax.numpy as jnp
from jax import lax
from jax.experimental import pallas as pl
from jax.experimental.pallas import tpu as pltpu
from jax.experimental.pallas import tpu_sc as plsc

_N = 10000
_E = 320000
_F_IN = 128
_H = 64
_C = 6

_NC, _NS = 2, 16
_CHUNK = 128
_CPT0 = 152
_CPT1 = 8
_CPTMAX = max(_CPT0, _CPT1)
_NCHUNKS_PAD = _NS * (_CPT0 + _CPT1)
_EPAD = _NCHUNKS_PAD * _CHUNK
_DUMP = 10000
_ROWS_PAD = 10240
_RPT = _ROWS_PAD // _NS
_NBUF = 4
_HBINS = 16384
_HSTRIP = _HBINS // _NS

_mesh = plsc.VectorSubcoreMesh(
    core_axis_name="c", subcore_axis_name="s",
    num_cores=_NC, num_subcores=_NS)


@functools.partial(
    pl.kernel,
    out_type=jax.ShapeDtypeStruct((_NC, _HBINS // 128, 128), jnp.float32),
    mesh=_mesh,
    scratch_types=[
        pltpu.VMEM((_CPTMAX, _CHUNK), jnp.int32),
        pltpu.VMEM((_HBINS,), jnp.float32),
        pltpu.VMEM((_HBINS,), jnp.float32),
        pltpu.VMEM((_HSTRIP // 128, 128), jnp.float32),
        pltpu.VMEM_SHARED((_NS, _HBINS), jnp.float32),
    ],
    compiler_params=pltpu.CompilerParams(needs_layout_passes=False),
)
def _sc_degree(edge_hbm, zerosh_hbm, out_hbm, idx_v, hist_v, buf_v, res_v,
               acc_sh):
    cid = lax.axis_index("c")
    sid = lax.axis_index("s")
    pltpu.sync_copy(zerosh_hbm, hist_v)
    ones16 = jnp.full((16,), 1.0, jnp.float32)

    def hist_part(cpt, base):
        pltpu.sync_copy(edge_hbm.at[1, pl.ds(base, cpt)],
                        idx_v.at[pl.ds(0, cpt)])
        for j in range(cpt):
            for k in range(_CHUNK // 16):
                d = idx_v[j, pl.ds(k * 16, 16)]
                plsc.addupdate_scatter(hist_v, [d], ones16)

    @pl.when(cid == 0)
    def _():
        hist_part(_CPT0, pl.multiple_of(sid * _CPT0, 8))

    if _CPT1 > 0:
        @pl.when(cid == 1)
        def _():
            hist_part(_CPT1,
                      pl.multiple_of(_NS * _CPT0 + sid * _CPT1, 8))

    pltpu.sync_copy(hist_v, acc_sh.at[sid])
    plsc.subcore_barrier()
    for r in range(_NS):
        pltpu.sync_copy(acc_sh.at[r, pl.ds(sid * _HSTRIP, _HSTRIP)],
                        buf_v.at[pl.ds(r * _HSTRIP, _HSTRIP)])
    for c in range(_HSTRIP // 16):
        acc = buf_v[pl.ds(c * 16, 16)]
        for r in range(1, _NS):
            acc = acc + buf_v[pl.ds(r * _HSTRIP + c * 16, 16)]
        res_v[c // 8, pl.ds((c % 8) * 16, 16)] = acc
    pltpu.sync_copy(res_v,
                    out_hbm.at[cid, pl.ds(sid * (_HSTRIP // 128),
                                          _HSTRIP // 128)])


@functools.partial(
    pl.kernel,
    out_type=jax.ShapeDtypeStruct((_NC, _ROWS_PAD, _H), jnp.float32),
    mesh=_mesh,
    scratch_types=(
        [pltpu.VMEM((_CPTMAX, _CHUNK), jnp.int32),
         pltpu.VMEM((_CPTMAX, _CHUNK), jnp.int32)]
        + [pltpu.VMEM((_CHUNK, _H), jnp.float32)] * _NBUF
        + [pltpu.SemaphoreType.DMA] * (2 * _NBUF)
        + [pltpu.VMEM_SHARED((_ROWS_PAD, _H), jnp.float32)]
    ),
    compiler_params=pltpu.CompilerParams(use_tc_tiling_on_sc=False),
)
def _sc_scatter(edge_hbm, g_hbm, zeros2_hbm, out_hbm, *rest):
    src_v, dst_v = rest[0], rest[1]
    rows = rest[2:2 + _NBUF]
    gsem = rest[2 + _NBUF:2 + 2 * _NBUF]
    ssem = rest[2 + 2 * _NBUF:2 + 3 * _NBUF]
    acc_sh = rest[2 + 3 * _NBUF]
    cid = lax.axis_index("c")
    sid = lax.axis_index("s")
    pltpu.sync_copy(zeros2_hbm, acc_sh.at[pl.ds(sid * _RPT, _RPT)])
    plsc.subcore_barrier()

    def ring(cpt, base):
        pltpu.sync_copy(edge_hbm.at[0, pl.ds(base, cpt)],
                        src_v.at[pl.ds(0, cpt)])
        pltpu.sync_copy(edge_hbm.at[1, pl.ds(base, cpt)],
                        dst_v.at[pl.ds(0, cpt)])
        gh = [None] * cpt
        sh = [None] * cpt
        for j in range(_NBUF - 1):
            gh[j] = pltpu.async_copy(g_hbm.at[src_v.at[j]], rows[j % _NBUF],
                                     gsem[j % _NBUF])
        for j in range(cpt):
            b = j % _NBUF
            jn = j + _NBUF - 1
            if jn < cpt:
                bn = jn % _NBUF
                if jn >= _NBUF:
                    sh[jn - _NBUF].wait()
                gh[jn] = pltpu.async_copy(g_hbm.at[src_v.at[jn]], rows[bn],
                                          gsem[bn])
            gh[j].wait()
            sh[j] = pltpu.async_copy(rows[b], acc_sh.at[dst_v.at[j]],
                                     ssem[b], add=True)
        for j in range(max(cpt - _NBUF, 0), cpt):
            sh[j].wait()

    @pl.when(cid == 0)
    def _():
        ring(_CPT0, pl.multiple_of(sid * _CPT0, 8))

    if _CPT1 > 0:
        @pl.when(cid == 1)
        def _():
            ring(_CPT1, pl.multiple_of(_NS * _CPT0 + sid * _CPT1, 8))

    plsc.subcore_barrier()
    pltpu.sync_copy(acc_sh.at[pl.ds(sid * _RPT, _RPT)],
                    out_hbm.at[cid, pl.ds(sid * _RPT, _RPT)])


_BLK = 1000
_GRID = _N // _BLK


def _stage1_body(degp_ref, x_ref, w1_ref, g1_ref, dis_ref):
    deg = degp_ref[:, 0:1] + degp_ref[:, 1:2] + 1.0
    dis = lax.rsqrt(deg)
    h = jnp.dot(x_ref[...], w1_ref[...], preferred_element_type=jnp.float32)
    g1_ref[...] = h * dis
    dis_ref[...] = dis


def _stage1(degp, x, w1):
    return pl.pallas_call(
        _stage1_body,
        grid=(_GRID,),
        in_specs=[
            pl.BlockSpec((_BLK, 2), lambda i: (i, 0)),
            pl.BlockSpec((_BLK, _F_IN), lambda i: (i, 0)),
            pl.BlockSpec((_F_IN, _H), lambda i: (0, 0)),
        ],
        out_specs=[
            pl.BlockSpec((_BLK, _H), lambda i: (i, 0)),
            pl.BlockSpec((_BLK, 1), lambda i: (i, 0)),
        ],
        out_shape=[
            jax.ShapeDtypeStruct((_N, _H), jnp.float32),
            jax.ShapeDtypeStruct((_N, 1), jnp.float32),
        ],
    )(degp, x, w1)


def _stage2_body(q_ref, g1_ref, dis_ref, b1_ref, w2_ref, g2_ref):
    s = q_ref[0] + q_ref[1] + g1_ref[...]
    h = jnp.maximum(s * dis_ref[...] + b1_ref[...], 0.0)
    h2 = jnp.dot(h, w2_ref[...], preferred_element_type=jnp.float32)
    g2_ref[...] = h2 * dis_ref[...]


def _stage2(q, g1, dis, b1, w2):
    return pl.pallas_call(
        _stage2_body,
        grid=(_GRID,),
        in_specs=[
            pl.BlockSpec((2, _BLK, _H), lambda i: (0, i, 0)),
            pl.BlockSpec((_BLK, _H), lambda i: (i, 0)),
            pl.BlockSpec((_BLK, 1), lambda i: (i, 0)),
            pl.BlockSpec((1, _H), lambda i: (0, 0)),
            pl.BlockSpec((_H, _H), lambda i: (0, 0)),
        ],
        out_specs=pl.BlockSpec((_BLK, _H), lambda i: (i, 0)),
        out_shape=jax.ShapeDtypeStruct((_N, _H), jnp.float32),
    )(q, g1, dis, b1, w2)


def _stage3_body(r_ref, g2_ref, dis_ref, b2_ref, wp_ref, bp_ref,
                 wc_ref, bc_ref, logits_ref, z_ref):
    s = r_ref[0] + r_ref[1] + g2_ref[...]
    h = jnp.maximum(s * dis_ref[...] + b2_ref[...], 0.0)
    z = jnp.maximum(
        jnp.dot(h, wp_ref[...], preferred_element_type=jnp.float32)
        + bp_ref[...], 0.0)
    z_ref[...] = z
    logits_ref[...] = (
        jnp.dot(z, wc_ref[...], preferred_element_type=jnp.float32)
        + bc_ref[...])


def _stage3(r, g2, dis, b2, wp, bp, wc, bc):
    return pl.pallas_call(
        _stage3_body,
        grid=(_GRID,),
        in_specs=[
            pl.BlockSpec((2, _BLK, _H), lambda i: (0, i, 0)),
            pl.BlockSpec((_BLK, _H), lambda i: (i, 0)),
            pl.BlockSpec((_BLK, 1), lambda i: (i, 0)),
            pl.BlockSpec((1, _H), lambda i: (0, 0)),
            pl.BlockSpec((_H, _H), lambda i: (0, 0)),
            pl.BlockSpec((1, _H), lambda i: (0, 0)),
            pl.BlockSpec((_H, _C), lambda i: (0, 0)),
            pl.BlockSpec((1, _C), lambda i: (0, 0)),
        ],
        out_specs=[
            pl.BlockSpec((_BLK, _C), lambda i: (i, 0)),
            pl.BlockSpec((_BLK, _H), lambda i: (i, 0)),
        ],
        out_shape=[
            jax.ShapeDtypeStruct((_N, _C), jnp.float32),
            jax.ShapeDtypeStruct((_N, _H), jnp.float32),
        ],
    )(r, g2, dis, b2, wp, bp, wc, bc)


def kernel(x, edge_index, W1, b1, W2, b2, Wp, bp, Wc, bc):
    zerosh = jnp.zeros((_HBINS,), jnp.float32)
    zeros2 = jnp.zeros((_RPT, _H), jnp.float32)

    pad = jnp.stack(
        [jnp.zeros((_EPAD - _E,), jnp.int32),
         _DUMP + (jnp.arange(_EPAD - _E, dtype=jnp.int32) % 240)], axis=0)
    e3 = jnp.concatenate([edge_index, pad], axis=1).reshape(
        2, _NCHUNKS_PAD, _CHUNK)

    degp = _sc_degree(e3, zerosh)
    degp = degp.reshape(_NC, _HBINS)[:, :_N].T

    g1, dis = _stage1(degp, x, W1)

    q = _sc_scatter(e3, g1, zeros2)
    g2 = _stage2(q, g1, dis, b1.reshape(1, _H), W2)

    r = _sc_scatter(e3, g2, zeros2)
    logits, z = _stage3(r, g2, dis, b2.reshape(1, _H), Wp,
                        bp.reshape(1, _H), Wc, bc.reshape(1, _C))
    return (logits, z)


# speedup vs baseline: 1.2680x; 1.0191x over previous
import functools

import jax
import jax.numpy as jnp
from jax import lax
from jax.experimental import pallas as pl
from jax.experimental.pallas import tpu as pltpu
from jax.experimental.pallas import tpu_sc as plsc

_N = 10000
_E = 320000
_F_IN = 128
_H = 64
_C = 6

_NC, _NS = 2, 16
_CHUNK = 128
_CPT0 = 152
_CPT1 = 8
_CPTMAX = max(_CPT0, _CPT1)
_NCHUNKS_PAD = _NS * (_CPT0 + _CPT1)
_EPAD = _NCHUNKS_PAD * _CHUNK
_DUMP = 10000
_ROWS_PAD = 10240
_RPT = _ROWS_PAD // _NS
_NBUF = 4
_HBINS = 16384
_HSTRIP = _HBINS // _NS

_mesh = plsc.VectorSubcoreMesh(
    core_axis_name="c", subcore_axis_name="s",
    num_cores=_NC, num_subcores=_NS)


@functools.partial(
    pl.kernel,
    out_type=jax.ShapeDtypeStruct((_NC, _HBINS // 128, 128), jnp.float32),
    mesh=_mesh,
    scratch_types=[
        pltpu.VMEM((_CPTMAX, _CHUNK), jnp.int32),
        pltpu.VMEM((_HBINS,), jnp.float32),
        pltpu.VMEM((_HBINS,), jnp.float32),
        pltpu.VMEM((_HSTRIP // 128, 128), jnp.float32),
        pltpu.VMEM_SHARED((_NS, _HBINS), jnp.float32),
    ],
    compiler_params=pltpu.CompilerParams(needs_layout_passes=False),
)
def _sc_degree(edge_hbm, zerosh_hbm, out_hbm, idx_v, hist_v, buf_v, res_v,
               acc_sh):
    cid = lax.axis_index("c")
    sid = lax.axis_index("s")
    pltpu.sync_copy(zerosh_hbm, hist_v)
    ones16 = jnp.full((16,), 1.0, jnp.float32)

    def hist_part(cpt, base):
        pltpu.sync_copy(edge_hbm.at[1, pl.ds(base, cpt)],
                        idx_v.at[pl.ds(0, cpt)])
        for j in range(cpt):
            for k in range(_CHUNK // 16):
                d = idx_v[j, pl.ds(k * 16, 16)]
                plsc.addupdate_scatter(hist_v, [d], ones16)

    @pl.when(cid == 0)
    def _():
        hist_part(_CPT0, pl.multiple_of(sid * _CPT0, 8))

    if _CPT1 > 0:
        @pl.when(cid == 1)
        def _():
            hist_part(_CPT1,
                      pl.multiple_of(_NS * _CPT0 + sid * _CPT1, 8))

    pltpu.sync_copy(hist_v, acc_sh.at[sid])
    plsc.subcore_barrier()
    for r in range(_NS):
        pltpu.sync_copy(acc_sh.at[r, pl.ds(sid * _HSTRIP, _HSTRIP)],
                        buf_v.at[pl.ds(r * _HSTRIP, _HSTRIP)])
    for c in range(_HSTRIP // 16):
        acc = buf_v[pl.ds(c * 16, 16)]
        for r in range(1, _NS):
            acc = acc + buf_v[pl.ds(r * _HSTRIP + c * 16, 16)]
        res_v[c // 8, pl.ds((c % 8) * 16, 16)] = acc
    pltpu.sync_copy(res_v,
                    out_hbm.at[cid, pl.ds(sid * (_HSTRIP // 128),
                                          _HSTRIP // 128)])


@functools.partial(
    pl.kernel,
    out_type=jax.ShapeDtypeStruct((_NC, _ROWS_PAD, _H), jnp.float32),
    mesh=_mesh,
    scratch_types=(
        [pltpu.VMEM((_CPTMAX, _CHUNK), jnp.int32),
         pltpu.VMEM((_CPTMAX, _CHUNK), jnp.int32)]
        + [pltpu.VMEM((_CHUNK, _H), jnp.float32)] * _NBUF
        + [pltpu.SemaphoreType.DMA] * (2 * _NBUF)
        + [pltpu.VMEM_SHARED((_ROWS_PAD, _H), jnp.float32)]
    ),
    compiler_params=pltpu.CompilerParams(use_tc_tiling_on_sc=False),
)
def _sc_scatter(edge_hbm, g_hbm, zeros2_hbm, out_hbm, *rest):
    src_v, dst_v = rest[0], rest[1]
    rows = rest[2:2 + _NBUF]
    gsem = rest[2 + _NBUF:2 + 2 * _NBUF]
    ssem = rest[2 + 2 * _NBUF:2 + 3 * _NBUF]
    acc_sh = rest[2 + 3 * _NBUF]
    cid = lax.axis_index("c")
    sid = lax.axis_index("s")
    pltpu.sync_copy(zeros2_hbm, acc_sh.at[pl.ds(sid * _RPT, _RPT)])
    plsc.subcore_barrier()

    def ring(cpt, base):
        pltpu.sync_copy(edge_hbm.at[0, pl.ds(base, cpt)],
                        src_v.at[pl.ds(0, cpt)])
        pltpu.sync_copy(edge_hbm.at[1, pl.ds(base, cpt)],
                        dst_v.at[pl.ds(0, cpt)])
        gh = [None] * cpt
        sh = [None] * cpt
        for j in range(_NBUF - 1):
            gh[j] = pltpu.async_copy(g_hbm.at[src_v.at[j]], rows[j % _NBUF],
                                     gsem[j % _NBUF])
        for j in range(cpt):
            b = j % _NBUF
            jn = j + _NBUF - 1
            if jn < cpt:
                bn = jn % _NBUF
                if jn >= _NBUF:
                    sh[jn - _NBUF].wait()
                gh[jn] = pltpu.async_copy(g_hbm.at[src_v.at[jn]], rows[bn],
                                          gsem[bn])
            gh[j].wait()
            sh[j] = pltpu.async_copy(rows[b], acc_sh.at[dst_v.at[j]],
                                     ssem[b], add=True)
        for j in range(max(cpt - _NBUF, 0), cpt):
            sh[j].wait()

    @pl.when(cid == 0)
    def _():
        ring(_CPT0, pl.multiple_of(sid * _CPT0, 8))

    if _CPT1 > 0:
        @pl.when(cid == 1)
        def _():
            ring(_CPT1, pl.multiple_of(_NS * _CPT0 + sid * _CPT1, 8))

    plsc.subcore_barrier()
    pltpu.sync_copy(acc_sh.at[pl.ds(sid * _RPT, _RPT)],
                    out_hbm.at[cid, pl.ds(sid * _RPT, _RPT)])


_BLK = 2000
_GRID = _N // _BLK


def _stage1_body(degp_ref, x_ref, w1_ref, g1_ref, dis_ref):
    deg = degp_ref[:, 0:1] + degp_ref[:, 1:2] + 1.0
    dis = lax.rsqrt(deg)
    h = jnp.dot(x_ref[...], w1_ref[...], preferred_element_type=jnp.float32)
    g1_ref[...] = h * dis
    dis_ref[...] = dis


def _stage1(degp, x, w1):
    return pl.pallas_call(
        _stage1_body,
        grid=(_GRID,),
        in_specs=[
            pl.BlockSpec((_BLK, 2), lambda i: (i, 0)),
            pl.BlockSpec((_BLK, _F_IN), lambda i: (i, 0)),
            pl.BlockSpec((_F_IN, _H), lambda i: (0, 0)),
        ],
        out_specs=[
            pl.BlockSpec((_BLK, _H), lambda i: (i, 0)),
            pl.BlockSpec((_BLK, 1), lambda i: (i, 0)),
        ],
        out_shape=[
            jax.ShapeDtypeStruct((_N, _H), jnp.float32),
            jax.ShapeDtypeStruct((_N, 1), jnp.float32),
        ],
    )(degp, x, w1)


def _stage2_body(q_ref, g1_ref, dis_ref, b1_ref, w2_ref, g2_ref):
    s = q_ref[0] + q_ref[1] + g1_ref[...]
    h = jnp.maximum(s * dis_ref[...] + b1_ref[...], 0.0)
    h2 = jnp.dot(h, w2_ref[...], preferred_element_type=jnp.float32)
    g2_ref[...] = h2 * dis_ref[...]


def _stage2(q, g1, dis, b1, w2):
    return pl.pallas_call(
        _stage2_body,
        grid=(_GRID,),
        in_specs=[
            pl.BlockSpec((2, _BLK, _H), lambda i: (0, i, 0)),
            pl.BlockSpec((_BLK, _H), lambda i: (i, 0)),
            pl.BlockSpec((_BLK, 1), lambda i: (i, 0)),
            pl.BlockSpec((1, _H), lambda i: (0, 0)),
            pl.BlockSpec((_H, _H), lambda i: (0, 0)),
        ],
        out_specs=pl.BlockSpec((_BLK, _H), lambda i: (i, 0)),
        out_shape=jax.ShapeDtypeStruct((_N, _H), jnp.float32),
    )(q, g1, dis, b1, w2)


def _stage3_body(r_ref, g2_ref, dis_ref, b2_ref, wp_ref, bp_ref,
                 wc_ref, bc_ref, logits_ref, z_ref):
    s = r_ref[0] + r_ref[1] + g2_ref[...]
    h = jnp.maximum(s * dis_ref[...] + b2_ref[...], 0.0)
    z = jnp.maximum(
        jnp.dot(h, wp_ref[...], preferred_element_type=jnp.float32)
        + bp_ref[...], 0.0)
    z_ref[...] = z
    logits_ref[...] = (
        jnp.dot(z, wc_ref[...], preferred_element_type=jnp.float32)
        + bc_ref[...])


def _stage3(r, g2, dis, b2, wp, bp, wc, bc):
    return pl.pallas_call(
        _stage3_body,
        grid=(_GRID,),
        in_specs=[
            pl.BlockSpec((2, _BLK, _H), lambda i: (0, i, 0)),
            pl.BlockSpec((_BLK, _H), lambda i: (i, 0)),
            pl.BlockSpec((_BLK, 1), lambda i: (i, 0)),
            pl.BlockSpec((1, _H), lambda i: (0, 0)),
            pl.BlockSpec((_H, _H), lambda i: (0, 0)),
            pl.BlockSpec((1, _H), lambda i: (0, 0)),
            pl.BlockSpec((_H, _C), lambda i: (0, 0)),
            pl.BlockSpec((1, _C), lambda i: (0, 0)),
        ],
        out_specs=[
            pl.BlockSpec((_BLK, _C), lambda i: (i, 0)),
            pl.BlockSpec((_BLK, _H), lambda i: (i, 0)),
        ],
        out_shape=[
            jax.ShapeDtypeStruct((_N, _C), jnp.float32),
            jax.ShapeDtypeStruct((_N, _H), jnp.float32),
        ],
    )(r, g2, dis, b2, wp, bp, wc, bc)


def kernel(x, edge_index, W1, b1, W2, b2, Wp, bp, Wc, bc):
    zerosh = jnp.zeros((_HBINS,), jnp.float32)
    zeros2 = jnp.zeros((_RPT, _H), jnp.float32)

    pad = jnp.stack(
        [jnp.zeros((_EPAD - _E,), jnp.int32),
         _DUMP + (jnp.arange(_EPAD - _E, dtype=jnp.int32) % 240)], axis=0)
    e3 = jnp.concatenate([edge_index, pad], axis=1).reshape(
        2, _NCHUNKS_PAD, _CHUNK)

    degp = _sc_degree(e3, zerosh)
    degp = degp.reshape(_NC, _HBINS)[:, :_N].T

    g1, dis = _stage1(degp, x, W1)

    q = _sc_scatter(e3, g1, zeros2)
    g2 = _stage2(q, g1, dis, b1.reshape(1, _H), W2)

    r = _sc_scatter(e3, g2, zeros2)
    logits, z = _stage3(r, g2, dis, b2.reshape(1, _H), Wp,
                        bp.reshape(1, _H), Wc, bc.reshape(1, _C))
    return (logits, z)
